# baseline (speedup 1.0000x reference)
import functools

import jax
import jax.numpy as jnp
from jax import lax
from jax.experimental import pallas as pl
from jax.experimental.pallas import tpu as pltpu
from jax.experimental.pallas import tpu_sc as plsc

N = 10000
E = 320000
F = 128
H = 128
DEPTH_ITERS = 4

NC = 2
NS = 16
NW = NC * NS
EPW = E // NW
CHUNK = 128
NCHUNK = 80
EPADW = NCHUNK * CHUNK
HALF = NCHUNK // 2
ACC_ROWS = N + 16
ROWS_A = 624
OUT_TAIL = N - NS * ROWS_A


def _agg_body(feat_hbm, src_hbm, dst_hbm, zeros_hbm, out_hbm,
              srcb_v, dstb_v, rows_v, acc_sh, gsem0, gsem1):
    gsems = (gsem0, gsem1)
    c = lax.axis_index("c")
    s = lax.axis_index("s")
    wid = s * NC + c
    tzero = c * 0

    off = pl.multiple_of(s * ROWS_A, 8)
    pltpu.sync_copy(zeros_hbm.at[pl.ds(off, ROWS_A)],
                    acc_sh.at[pl.ds(off, ROWS_A)])

    @pl.when(s == NS - 1)
    def _zero_tail():
        pltpu.sync_copy(zeros_hbm.at[pl.ds(NS * ROWS_A, ACC_ROWS - NS * ROWS_A)],
                        acc_sh.at[pl.ds(NS * ROWS_A, ACC_ROWS - NS * ROWS_A)])

    pltpu.sync_copy(src_hbm.at[wid, 0], srcb_v)
    pltpu.sync_copy(dst_hbm.at[wid, 0], dstb_v)
    plsc.subcore_barrier()

    def gather(q, b):
        return pltpu.async_copy(feat_hbm.at[srcb_v.at[q + tzero]],
                                rows_v.at[b], gsems[b])

    for half in range(2):
        if half == 1:
            pltpu.sync_copy(src_hbm.at[wid, 1], srcb_v)
            pltpu.sync_copy(dst_hbm.at[wid, 1], dstb_v)
        g = [gather(0, 0), None]
        for q in range(HALF):
            b = q % 2
            g[b].wait()
            if q + 1 < HALF:
                g[1 - b] = gather(q + 1, 1 - b)
            pltpu.sync_copy(rows_v.at[b], acc_sh.at[dstb_v.at[q + tzero]],
                            add=True)

    plsc.subcore_barrier()

    pltpu.sync_copy(acc_sh.at[pl.ds(off, ROWS_A)],
                    out_hbm.at[c, pl.ds(off, ROWS_A)])

    @pl.when(s == NS - 1)
    def _out_tail():
        pltpu.sync_copy(acc_sh.at[pl.ds(NS * ROWS_A, OUT_TAIL)],
                        out_hbm.at[c, pl.ds(NS * ROWS_A, OUT_TAIL)])


_agg = pl.kernel(
    _agg_body,
    out_type=jax.ShapeDtypeStruct((NC, N, F), jnp.float32),
    mesh=plsc.VectorSubcoreMesh(core_axis_name="c", subcore_axis_name="s"),
    scratch_types=[
        pltpu.VMEM((HALF, CHUNK), jnp.int32),
        pltpu.VMEM((HALF, CHUNK), jnp.int32),
        pltpu.VMEM((2, CHUNK, F), jnp.float32),
        pltpu.VMEM_SHARED((ACC_ROWS, F), jnp.float32),
        pltpu.SemaphoreType.DMA,
        pltpu.SemaphoreType.DMA,
    ],
)


_ROWS = 1000
_GRID = N // _ROWS


def _lin_relu_tc(p_ref, w_ref, b_ref, o_ref):
    a = p_ref[0] + p_ref[1]
    z = lax.dot_general(a, w_ref[...], (((1,), (1,)), ((), ())),
                        preferred_element_type=jnp.float32)
    o_ref[...] = jnp.maximum(z + b_ref[...], 0.0)


def _lstm_tc(p_ref, h_ref, c_ref, wih_ref, whh_ref, b_ref, ho_ref, co_ref):
    a = p_ref[0] + p_ref[1]
    g = (lax.dot_general(a, wih_ref[...], (((1,), (1,)), ((), ())),
                         preferred_element_type=jnp.float32)
         + lax.dot_general(h_ref[...], whh_ref[...], (((1,), (1,)), ((), ())),
                           preferred_element_type=jnp.float32)
         + b_ref[...])
    i = jax.nn.sigmoid(g[:, 0:H])
    f = jax.nn.sigmoid(g[:, H:2 * H])
    gg = jnp.tanh(g[:, 2 * H:3 * H])
    o = jax.nn.sigmoid(g[:, 3 * H:4 * H])
    cc = f * c_ref[...] + i * gg
    ho_ref[...] = o * jnp.tanh(cc)
    co_ref[...] = cc


def _out_tc(p_ref, w_ref, b_ref, o_ref):
    a = p_ref[0] + p_ref[1]
    o_ref[...] = lax.dot_general(a, w_ref[...], (((1,), (1,)), ((), ())),
                                 preferred_element_type=jnp.float32) + b_ref[...]


_lin_relu = pl.pallas_call(
    _lin_relu_tc,
    grid=(_GRID,),
    in_specs=[
        pl.BlockSpec((2, _ROWS, F), lambda i: (0, i, 0)),
        pl.BlockSpec((H, F), lambda i: (0, 0)),
        pl.BlockSpec((1, H), lambda i: (0, 0)),
    ],
    out_specs=pl.BlockSpec((_ROWS, H), lambda i: (i, 0)),
    out_shape=jax.ShapeDtypeStruct((N, H), jnp.float32),
)

_lstm = pl.pallas_call(
    _lstm_tc,
    grid=(_GRID,),
    in_specs=[
        pl.BlockSpec((2, _ROWS, H), lambda i: (0, i, 0)),
        pl.BlockSpec((_ROWS, H), lambda i: (i, 0)),
        pl.BlockSpec((_ROWS, H), lambda i: (i, 0)),
        pl.BlockSpec((4 * H, H), lambda i: (0, 0)),
        pl.BlockSpec((4 * H, H), lambda i: (0, 0)),
        pl.BlockSpec((1, 4 * H), lambda i: (0, 0)),
    ],
    out_specs=[
        pl.BlockSpec((_ROWS, H), lambda i: (i, 0)),
        pl.BlockSpec((_ROWS, H), lambda i: (i, 0)),
    ],
    out_shape=[
        jax.ShapeDtypeStruct((N, H), jnp.float32),
        jax.ShapeDtypeStruct((N, H), jnp.float32),
    ],
)

_linear_out = pl.pallas_call(
    _out_tc,
    grid=(_GRID,),
    in_specs=[
        pl.BlockSpec((2, _ROWS, H), lambda i: (0, i, 0)),
        pl.BlockSpec((128, H), lambda i: (0, 0)),
        pl.BlockSpec((1, 128), lambda i: (0, 0)),
    ],
    out_specs=pl.BlockSpec((_ROWS, 128), lambda i: (i, 0)),
    out_shape=jax.ShapeDtypeStruct((N, 128), jnp.float32),
)


def kernel(features, edge_index, W_in, b_in, W_ih, W_hh, b_ih, b_hh, W_out, b_out):
    pad = EPADW - EPW
    src = jnp.pad(edge_index[0].reshape(NW, EPW),
                  ((0, 0), (0, pad))).reshape(NW, 2, HALF, CHUNK)
    dst = jnp.pad(edge_index[1].reshape(NW, EPW), ((0, 0), (0, pad)),
                  constant_values=N).reshape(NW, 2, HALF, CHUNK)
    zeros = jnp.zeros((ACC_ROWS, F), jnp.float32)
    b_in2 = b_in.reshape(1, H)
    b_g = (b_ih + b_hh).reshape(1, 4 * H)
    W_out_pad = jnp.zeros((128, H), jnp.float32).at[0].set(W_out[0])
    b_o = jnp.zeros((1, 128), jnp.float32).at[0, 0].set(b_out[0])

    p = _agg(features, src, dst, zeros)
    h = _lin_relu(p, W_in, b_in2)

    h_t = jnp.zeros((N, H), jnp.float32)
    c_t = jnp.zeros((N, H), jnp.float32)

    p = _agg(h, src, dst, zeros)
    h_t, c_t = _lstm(p, h_t, c_t, W_ih, W_hh, b_g)
    for _ in range(DEPTH_ITERS):
        p = _agg(h_t, src, dst, zeros)
        h_t, c_t = _lstm(p, h_t, c_t, W_ih, W_hh, b_g)

    p = _agg(h_t, src, dst, zeros)
    return _linear_out(p, W_out_pad, b_o)[:, :1]


# speedup vs baseline: 1.0037x; 1.0037x over previous
import functools

import jax
import jax.numpy as jnp
from jax import lax
from jax.experimental import pallas as pl
from jax.experimental.pallas import tpu as pltpu
from jax.experimental.pallas import tpu_sc as plsc

N = 10000
E = 320000
F = 128
H = 128
DEPTH_ITERS = 4

NC = 2
NS = 16
NW = NC * NS
EPW = E // NW
CHUNK = 128
NCHUNK = 80
EPADW = NCHUNK * CHUNK
HALF = NCHUNK // 2
ACC_ROWS = N + 16
ROWS_A = 624
OUT_TAIL = N - NS * ROWS_A


def _agg_body(feat_hbm, src_hbm, dst_hbm, zeros_hbm, out_hbm,
              srcb_v, dstb_v, rows_v, acc_sh, gsem0, gsem1):
    gsems = (gsem0, gsem1)
    c = lax.axis_index("c")
    s = lax.axis_index("s")
    wid = s * NC + c
    tzero = c * 0

    off = pl.multiple_of(s * ROWS_A, 8)
    pltpu.sync_copy(zeros_hbm.at[pl.ds(off, ROWS_A)],
                    acc_sh.at[pl.ds(off, ROWS_A)])

    @pl.when(s == NS - 1)
    def _zero_tail():
        pltpu.sync_copy(zeros_hbm.at[pl.ds(NS * ROWS_A, ACC_ROWS - NS * ROWS_A)],
                        acc_sh.at[pl.ds(NS * ROWS_A, ACC_ROWS - NS * ROWS_A)])

    pltpu.sync_copy(src_hbm.at[wid, 0], srcb_v)
    pltpu.sync_copy(dst_hbm.at[wid, 0], dstb_v)
    plsc.subcore_barrier()

    def gather(q, b):
        return pltpu.async_copy(feat_hbm.at[srcb_v.at[q + tzero]],
                                rows_v.at[b], gsems[b])

    for half in range(2):
        if half == 1:
            pltpu.sync_copy(src_hbm.at[wid, 1], srcb_v)
            pltpu.sync_copy(dst_hbm.at[wid, 1], dstb_v)
        g = [gather(0, 0), None]
        for q in range(HALF):
            b = q % 2
            g[b].wait()
            if q + 1 < HALF:
                g[1 - b] = gather(q + 1, 1 - b)
            pltpu.sync_copy(rows_v.at[b], acc_sh.at[dstb_v.at[q + tzero]],
                            add=True)

    plsc.subcore_barrier()

    pltpu.sync_copy(acc_sh.at[pl.ds(off, ROWS_A)],
                    out_hbm.at[c, pl.ds(off, ROWS_A)])

    @pl.when(s == NS - 1)
    def _out_tail():
        pltpu.sync_copy(acc_sh.at[pl.ds(NS * ROWS_A, OUT_TAIL)],
                        out_hbm.at[c, pl.ds(NS * ROWS_A, OUT_TAIL)])


_agg = pl.kernel(
    _agg_body,
    out_type=jax.ShapeDtypeStruct((NC, N, F), jnp.float32),
    mesh=plsc.VectorSubcoreMesh(core_axis_name="c", subcore_axis_name="s"),
    scratch_types=[
        pltpu.VMEM((HALF, CHUNK), jnp.int32),
        pltpu.VMEM((HALF, CHUNK), jnp.int32),
        pltpu.VMEM((2, CHUNK, F), jnp.float32),
        pltpu.VMEM_SHARED((ACC_ROWS, F), jnp.float32),
        pltpu.SemaphoreType.DMA,
        pltpu.SemaphoreType.DMA,
    ],
)


_ROWS = 1000
_GRID = N // _ROWS


def _lin_relu_tc(p_ref, w_ref, b_ref, o_ref):
    a = p_ref[0] + p_ref[1]
    z = lax.dot_general(a, w_ref[...], (((1,), (1,)), ((), ())),
                        preferred_element_type=jnp.float32)
    o_ref[...] = jnp.maximum(z + b_ref[...], 0.0)


def _lstm_tc(p_ref, h_ref, c_ref, wih_ref, whh_ref, b_ref, ho_ref, co_ref):
    a = p_ref[0] + p_ref[1]
    g = (lax.dot_general(a, wih_ref[...], (((1,), (1,)), ((), ())),
                         preferred_element_type=jnp.float32)
         + lax.dot_general(h_ref[...], whh_ref[...], (((1,), (1,)), ((), ())),
                           preferred_element_type=jnp.float32)
         + b_ref[...])
    i = jax.nn.sigmoid(g[:, 0:H])
    f = jax.nn.sigmoid(g[:, H:2 * H])
    gg = jnp.tanh(g[:, 2 * H:3 * H])
    o = jax.nn.sigmoid(g[:, 3 * H:4 * H])
    cc = f * c_ref[...] + i * gg
    ho_ref[...] = o * jnp.tanh(cc)
    co_ref[...] = cc


def _out_tc(p_ref, w_ref, b_ref, o_ref):
    a = p_ref[0] + p_ref[1]
    o_ref[...] = lax.dot_general(a, w_ref[...], (((1,), (1,)), ((), ())),
                                 preferred_element_type=jnp.float32) + b_ref[...]


_lin_relu = pl.pallas_call(
    _lin_relu_tc,
    grid=(_GRID,),
    in_specs=[
        pl.BlockSpec((2, _ROWS, F), lambda i: (0, i, 0)),
        pl.BlockSpec((H, F), lambda i: (0, 0)),
        pl.BlockSpec((1, H), lambda i: (0, 0)),
    ],
    out_specs=pl.BlockSpec((_ROWS, H), lambda i: (i, 0)),
    out_shape=jax.ShapeDtypeStruct((N, H), jnp.float32),
)

_lstm = pl.pallas_call(
    _lstm_tc,
    grid=(_GRID,),
    in_specs=[
        pl.BlockSpec((2, _ROWS, H), lambda i: (0, i, 0)),
        pl.BlockSpec((_ROWS, H), lambda i: (i, 0)),
        pl.BlockSpec((_ROWS, H), lambda i: (i, 0)),
        pl.BlockSpec((4 * H, H), lambda i: (0, 0)),
        pl.BlockSpec((4 * H, H), lambda i: (0, 0)),
        pl.BlockSpec((1, 4 * H), lambda i: (0, 0)),
    ],
    out_specs=[
        pl.BlockSpec((_ROWS, H), lambda i: (i, 0)),
        pl.BlockSpec((_ROWS, H), lambda i: (i, 0)),
    ],
    out_shape=[
        jax.ShapeDtypeStruct((N, H), jnp.float32),
        jax.ShapeDtypeStruct((N, H), jnp.float32),
    ],
)

_linear_out = pl.pallas_call(
    _out_tc,
    grid=(_GRID,),
    in_specs=[
        pl.BlockSpec((2, _ROWS, H), lambda i: (0, i, 0)),
        pl.BlockSpec((128, H), lambda i: (0, 0)),
        pl.BlockSpec((1, 128), lambda i: (0, 0)),
    ],
    out_specs=pl.BlockSpec((_ROWS, 128), lambda i: (i, 0)),
    out_shape=jax.ShapeDtypeStruct((N, 128), jnp.float32),
)


def kernel(features, edge_index, W_in, b_in, W_ih, W_hh, b_ih, b_hh, W_out, b_out):
    pad = EPADW - EPW
    src = jnp.pad(edge_index[0].reshape(NW, EPW),
                  ((0, 0), (0, pad))).reshape(NW, 2, HALF, CHUNK)
    dummy_row = N + (jnp.arange(NW, dtype=jnp.int32) // NC)
    dst = jnp.concatenate(
        [edge_index[1].reshape(NW, EPW),
         jnp.broadcast_to(dummy_row[:, None], (NW, pad))],
        axis=1).reshape(NW, 2, HALF, CHUNK)
    zeros = jnp.zeros((ACC_ROWS, F), jnp.float32)
    b_in2 = b_in.reshape(1, H)
    b_g = (b_ih + b_hh).reshape(1, 4 * H)
    W_out_pad = jnp.zeros((128, H), jnp.float32).at[0].set(W_out[0])
    b_o = jnp.zeros((1, 128), jnp.float32).at[0, 0].set(b_out[0])

    p = _agg(features, src, dst, zeros)
    h = _lin_relu(p, W_in, b_in2)

    h_t = jnp.zeros((N, H), jnp.float32)
    c_t = jnp.zeros((N, H), jnp.float32)

    p = _agg(h, src, dst, zeros)
    h_t, c_t = _lstm(p, h_t, c_t, W_ih, W_hh, b_g)
    for _ in range(DEPTH_ITERS):
        p = _agg(h_t, src, dst, zeros)
        h_t, c_t = _lstm(p, h_t, c_t, W_ih, W_hh, b_g)

    p = _agg(h_t, src, dst, zeros)
    return _linear_out(p, W_out_pad, b_o)[:, :1]


# speedup vs baseline: 2.2388x; 2.2305x over previous
import functools

import jax
import jax.numpy as jnp
from jax import lax
from jax.experimental import pallas as pl
from jax.experimental.pallas import tpu as pltpu
from jax.experimental.pallas import tpu_sc as plsc

N = 10000
E = 320000
F = 128
H = 128
DEPTH_ITERS = 4

NC = 2
NS = 16
NW = NC * NS
EPW = E // NW
CHUNK = 125
NCHUNK = EPW // CHUNK
ROWS_A = 624
ROWS_REM = N - NS * ROWS_A


def _agg_body(feat_hbm, src_hbm, dst_hbm, zeros_hbm, out_hbm,
              src_v, dst_v, rows_v, acc_sh, gsem):
    c = lax.axis_index("c")
    s = lax.axis_index("s")
    wid = s * NC + c

    off = pl.multiple_of(s * ROWS_A, 8)
    pltpu.sync_copy(zeros_hbm.at[pl.ds(off, ROWS_A)],
                    acc_sh.at[pl.ds(off, ROWS_A)])

    @pl.when(s == NS - 1)
    def _zero_tail():
        pltpu.sync_copy(zeros_hbm.at[pl.ds(NS * ROWS_A, ROWS_REM)],
                        acc_sh.at[pl.ds(NS * ROWS_A, ROWS_REM)])

    pltpu.sync_copy(src_hbm.at[wid], src_v)
    pltpu.sync_copy(dst_hbm.at[wid], dst_v)
    plsc.subcore_barrier()

    def body(j, carry):
        pltpu.async_copy(feat_hbm.at[src_v.at[j]], rows_v, gsem).wait()
        pltpu.sync_copy(rows_v, acc_sh.at[dst_v.at[j]], add=True)
        return carry

    lax.fori_loop(0, NCHUNK, body, 0)
    plsc.subcore_barrier()
    pltpu.sync_copy(acc_sh.at[pl.ds(off, ROWS_A)],
                    out_hbm.at[c, pl.ds(off, ROWS_A)])

    @pl.when(s == NS - 1)
    def _out_tail():
        pltpu.sync_copy(acc_sh.at[pl.ds(NS * ROWS_A, ROWS_REM)],
                        out_hbm.at[c, pl.ds(NS * ROWS_A, ROWS_REM)])


_agg = pl.kernel(
    _agg_body,
    out_type=jax.ShapeDtypeStruct((NC, N, F), jnp.float32),
    mesh=plsc.VectorSubcoreMesh(core_axis_name="c", subcore_axis_name="s"),
    scratch_types=[
        pltpu.VMEM((NCHUNK, CHUNK), jnp.int32),
        pltpu.VMEM((NCHUNK, CHUNK), jnp.int32),
        pltpu.VMEM((CHUNK, F), jnp.float32),
        pltpu.VMEM_SHARED((N, F), jnp.float32),
        pltpu.SemaphoreType.DMA,
    ],
)


_ROWS = 1000
_GRID = N // _ROWS


def _lin_relu_tc(p_ref, w_ref, b_ref, o_ref):
    a = p_ref[0] + p_ref[1]
    z = lax.dot_general(a, w_ref[...], (((1,), (1,)), ((), ())),
                        preferred_element_type=jnp.float32)
    o_ref[...] = jnp.maximum(z + b_ref[...], 0.0)


def _lstm_tc(p_ref, h_ref, c_ref, wih_ref, whh_ref, b_ref, ho_ref, co_ref):
    a = p_ref[0] + p_ref[1]
    g = (lax.dot_general(a, wih_ref[...], (((1,), (1,)), ((), ())),
                         preferred_element_type=jnp.float32)
         + lax.dot_general(h_ref[...], whh_ref[...], (((1,), (1,)), ((), ())),
                           preferred_element_type=jnp.float32)
         + b_ref[...])
    i = jax.nn.sigmoid(g[:, 0:H])
    f = jax.nn.sigmoid(g[:, H:2 * H])
    gg = jnp.tanh(g[:, 2 * H:3 * H])
    o = jax.nn.sigmoid(g[:, 3 * H:4 * H])
    cc = f * c_ref[...] + i * gg
    ho_ref[...] = o * jnp.tanh(cc)
    co_ref[...] = cc


def _out_tc(p_ref, w_ref, b_ref, o_ref):
    a = p_ref[0] + p_ref[1]
    o_ref[...] = lax.dot_general(a, w_ref[...], (((1,), (1,)), ((), ())),
                                 preferred_element_type=jnp.float32) + b_ref[...]


_lin_relu = pl.pallas_call(
    _lin_relu_tc,
    grid=(_GRID,),
    in_specs=[
        pl.BlockSpec((2, _ROWS, F), lambda i: (0, i, 0)),
        pl.BlockSpec((H, F), lambda i: (0, 0)),
        pl.BlockSpec((1, H), lambda i: (0, 0)),
    ],
    out_specs=pl.BlockSpec((_ROWS, H), lambda i: (i, 0)),
    out_shape=jax.ShapeDtypeStruct((N, H), jnp.float32),
)

_lstm = pl.pallas_call(
    _lstm_tc,
    grid=(_GRID,),
    in_specs=[
        pl.BlockSpec((2, _ROWS, H), lambda i: (0, i, 0)),
        pl.BlockSpec((_ROWS, H), lambda i: (i, 0)),
        pl.BlockSpec((_ROWS, H), lambda i: (i, 0)),
        pl.BlockSpec((4 * H, H), lambda i: (0, 0)),
        pl.BlockSpec((4 * H, H), lambda i: (0, 0)),
        pl.BlockSpec((1, 4 * H), lambda i: (0, 0)),
    ],
    out_specs=[
        pl.BlockSpec((_ROWS, H), lambda i: (i, 0)),
        pl.BlockSpec((_ROWS, H), lambda i: (i, 0)),
    ],
    out_shape=[
        jax.ShapeDtypeStruct((N, H), jnp.float32),
        jax.ShapeDtypeStruct((N, H), jnp.float32),
    ],
)

_linear_out = pl.pallas_call(
    _out_tc,
    grid=(_GRID,),
    in_specs=[
        pl.BlockSpec((2, _ROWS, H), lambda i: (0, i, 0)),
        pl.BlockSpec((128, H), lambda i: (0, 0)),
        pl.BlockSpec((1, 128), lambda i: (0, 0)),
    ],
    out_specs=pl.BlockSpec((_ROWS, 128), lambda i: (i, 0)),
    out_shape=jax.ShapeDtypeStruct((N, 128), jnp.float32),
)


def kernel(features, edge_index, W_in, b_in, W_ih, W_hh, b_ih, b_hh, W_out, b_out):
    src = edge_index[0].reshape(NW, NCHUNK, CHUNK)
    dst = edge_index[1].reshape(NW, NCHUNK, CHUNK)
    zeros = jnp.zeros((N, F), jnp.float32)
    b_in2 = b_in.reshape(1, H)
    b_g = (b_ih + b_hh).reshape(1, 4 * H)
    W_out_pad = jnp.zeros((128, H), jnp.float32).at[0].set(W_out[0])
    b_o = jnp.zeros((1, 128), jnp.float32).at[0, 0].set(b_out[0])

    p = _agg(features, src, dst, zeros)
    h = _lin_relu(p, W_in, b_in2)

    h_t = jnp.zeros((N, H), jnp.float32)
    c_t = jnp.zeros((N, H), jnp.float32)

    p = _agg(h, src, dst, zeros)
    h_t, c_t = _lstm(p, h_t, c_t, W_ih, W_hh, b_g)
    for _ in range(DEPTH_ITERS):
        p = _agg(h_t, src, dst, zeros)
        h_t, c_t = _lstm(p, h_t, c_t, W_ih, W_hh, b_g)

    p = _agg(h_t, src, dst, zeros)
    return _linear_out(p, W_out_pad, b_o)[:, :1]


# speedup vs baseline: 3.3718x; 1.5061x over previous
import functools

import jax
import jax.numpy as jnp
from jax import lax
from jax.experimental import pallas as pl
from jax.experimental.pallas import tpu as pltpu
from jax.experimental.pallas import tpu_sc as plsc

N = 10000
E = 320000
F = 128
H = 128
DEPTH_ITERS = 4

NC = 2
NS = 16
NW = NC * NS
EPW = E // NW
CHUNK = 125
NCHUNK = EPW // CHUNK
HALF = NCHUNK // 2
ROWS_A = 624
ROWS_REM = N - NS * ROWS_A


def _agg_body(feat_hbm, src_hbm, dst_hbm, zeros_hbm, out_hbm,
              src_v, dst_v, rows_v, acc_sh, gsem0, gsem1):
    gsems = (gsem0, gsem1)
    c = lax.axis_index("c")
    s = lax.axis_index("s")
    wid = s * NC + c

    off = pl.multiple_of(s * ROWS_A, 8)
    pltpu.sync_copy(zeros_hbm.at[pl.ds(off, ROWS_A)],
                    acc_sh.at[pl.ds(off, ROWS_A)])

    @pl.when(s == NS - 1)
    def _zero_tail():
        pltpu.sync_copy(zeros_hbm.at[pl.ds(NS * ROWS_A, ROWS_REM)],
                        acc_sh.at[pl.ds(NS * ROWS_A, ROWS_REM)])

    pltpu.sync_copy(src_hbm.at[wid, 0], src_v)
    pltpu.sync_copy(dst_hbm.at[wid, 0], dst_v)
    plsc.subcore_barrier()

    def gather(q, b):
        return pltpu.async_copy(feat_hbm.at[src_v.at[q]], rows_v.at[b],
                                gsems[b])

    def wait_gather(b):
        pltpu.make_async_copy(feat_hbm.at[src_v.at[0]], rows_v.at[b],
                              gsems[b]).wait()

    def scatter(q, b):
        pltpu.sync_copy(rows_v.at[b], acc_sh.at[dst_v.at[q]], add=True)

    for half in range(2):
        if half == 1:
            pltpu.sync_copy(src_hbm.at[wid, 1], src_v)
            pltpu.sync_copy(dst_hbm.at[wid, 1], dst_v)
        gather(0, 0)

        def body(t, carry):
            gather(2 * t + 1, 1)
            wait_gather(0)
            scatter(2 * t, 0)
            gather(2 * t + 2, 0)
            wait_gather(1)
            scatter(2 * t + 1, 1)
            return carry

        lax.fori_loop(0, HALF // 2 - 1, body, 0)
        gather(HALF - 1, 1)
        wait_gather(0)
        scatter(HALF - 2, 0)
        wait_gather(1)
        scatter(HALF - 1, 1)

    plsc.subcore_barrier()
    pltpu.sync_copy(acc_sh.at[pl.ds(off, ROWS_A)],
                    out_hbm.at[c, pl.ds(off, ROWS_A)])

    @pl.when(s == NS - 1)
    def _out_tail():
        pltpu.sync_copy(acc_sh.at[pl.ds(NS * ROWS_A, ROWS_REM)],
                        out_hbm.at[c, pl.ds(NS * ROWS_A, ROWS_REM)])


_agg = pl.kernel(
    _agg_body,
    out_type=jax.ShapeDtypeStruct((NC, N, F), jnp.float32),
    mesh=plsc.VectorSubcoreMesh(core_axis_name="c", subcore_axis_name="s"),
    scratch_types=[
        pltpu.VMEM((HALF, CHUNK), jnp.int32),
        pltpu.VMEM((HALF, CHUNK), jnp.int32),
        pltpu.VMEM((2, CHUNK, F), jnp.float32),
        pltpu.VMEM_SHARED((N, F), jnp.float32),
        pltpu.SemaphoreType.DMA,
        pltpu.SemaphoreType.DMA,
    ],
)


_ROWS = 1000
_GRID = N // _ROWS


def _lin_relu_tc(p_ref, w_ref, b_ref, o_ref):
    a = p_ref[0] + p_ref[1]
    z = lax.dot_general(a, w_ref[...], (((1,), (1,)), ((), ())),
                        preferred_element_type=jnp.float32)
    o_ref[...] = jnp.maximum(z + b_ref[...], 0.0)


def _lstm_tc(p_ref, h_ref, c_ref, wih_ref, whh_ref, b_ref, ho_ref, co_ref):
    a = p_ref[0] + p_ref[1]
    g = (lax.dot_general(a, wih_ref[...], (((1,), (1,)), ((), ())),
                         preferred_element_type=jnp.float32)
         + lax.dot_general(h_ref[...], whh_ref[...], (((1,), (1,)), ((), ())),
                           preferred_element_type=jnp.float32)
         + b_ref[...])
    i = jax.nn.sigmoid(g[:, 0:H])
    f = jax.nn.sigmoid(g[:, H:2 * H])
    gg = jnp.tanh(g[:, 2 * H:3 * H])
    o = jax.nn.sigmoid(g[:, 3 * H:4 * H])
    cc = f * c_ref[...] + i * gg
    ho_ref[...] = o * jnp.tanh(cc)
    co_ref[...] = cc


def _out_tc(p_ref, w_ref, b_ref, o_ref):
    a = p_ref[0] + p_ref[1]
    o_ref[...] = lax.dot_general(a, w_ref[...], (((1,), (1,)), ((), ())),
                                 preferred_element_type=jnp.float32) + b_ref[...]


_lin_relu = pl.pallas_call(
    _lin_relu_tc,
    grid=(_GRID,),
    in_specs=[
        pl.BlockSpec((2, _ROWS, F), lambda i: (0, i, 0)),
        pl.BlockSpec((H, F), lambda i: (0, 0)),
        pl.BlockSpec((1, H), lambda i: (0, 0)),
    ],
    out_specs=pl.BlockSpec((_ROWS, H), lambda i: (i, 0)),
    out_shape=jax.ShapeDtypeStruct((N, H), jnp.float32),
)

_lstm = pl.pallas_call(
    _lstm_tc,
    grid=(_GRID,),
    in_specs=[
        pl.BlockSpec((2, _ROWS, H), lambda i: (0, i, 0)),
        pl.BlockSpec((_ROWS, H), lambda i: (i, 0)),
        pl.BlockSpec((_ROWS, H), lambda i: (i, 0)),
        pl.BlockSpec((4 * H, H), lambda i: (0, 0)),
        pl.BlockSpec((4 * H, H), lambda i: (0, 0)),
        pl.BlockSpec((1, 4 * H), lambda i: (0, 0)),
    ],
    out_specs=[
        pl.BlockSpec((_ROWS, H), lambda i: (i, 0)),
        pl.BlockSpec((_ROWS, H), lambda i: (i, 0)),
    ],
    out_shape=[
        jax.ShapeDtypeStruct((N, H), jnp.float32),
        jax.ShapeDtypeStruct((N, H), jnp.float32),
    ],
)

_linear_out = pl.pallas_call(
    _out_tc,
    grid=(_GRID,),
    in_specs=[
        pl.BlockSpec((2, _ROWS, H), lambda i: (0, i, 0)),
        pl.BlockSpec((128, H), lambda i: (0, 0)),
        pl.BlockSpec((1, 128), lambda i: (0, 0)),
    ],
    out_specs=pl.BlockSpec((_ROWS, 128), lambda i: (i, 0)),
    out_shape=jax.ShapeDtypeStruct((N, 128), jnp.float32),
)


def kernel(features, edge_index, W_in, b_in, W_ih, W_hh, b_ih, b_hh, W_out, b_out):
    src = edge_index[0].reshape(NW, 2, HALF, CHUNK)
    dst = edge_index[1].reshape(NW, 2, HALF, CHUNK)
    zeros = jnp.zeros((N, F), jnp.float32)
    b_in2 = b_in.reshape(1, H)
    b_g = (b_ih + b_hh).reshape(1, 4 * H)
    W_out_pad = jnp.zeros((128, H), jnp.float32).at[0].set(W_out[0])
    b_o = jnp.zeros((1, 128), jnp.float32).at[0, 0].set(b_out[0])

    p = _agg(features, src, dst, zeros)
    h = _lin_relu(p, W_in, b_in2)

    h_t = jnp.zeros((N, H), jnp.float32)
    c_t = jnp.zeros((N, H), jnp.float32)

    p = _agg(h, src, dst, zeros)
    h_t, c_t = _lstm(p, h_t, c_t, W_ih, W_hh, b_g)
    for _ in range(DEPTH_ITERS):
        p = _agg(h_t, src, dst, zeros)
        h_t, c_t = _lstm(p, h_t, c_t, W_ih, W_hh, b_g)

    p = _agg(h_t, src, dst, zeros)
    return _linear_out(p, W_out_pad, b_o)[:, :1]


# speedup vs baseline: 3.4984x; 1.0376x over previous
import functools

import jax
import jax.numpy as jnp
from jax import lax
from jax.experimental import pallas as pl
from jax.experimental.pallas import tpu as pltpu
from jax.experimental.pallas import tpu_sc as plsc

N = 10000
E = 320000
F = 128
H = 128
DEPTH_ITERS = 4

NC = 2
NS = 16
NW = NC * NS
EPW = E // NW
CHUNK = 125
NCHUNK = EPW // CHUNK
HALF = NCHUNK // 2
ROWS_A = 624
ROWS_REM = N - NS * ROWS_A


def _agg_body(feat_hbm, src_hbm, dst_hbm, zeros_hbm, out_hbm,
              src_v, dst_v, rows_v, acc_sh, gsem0, gsem1):
    gsems = (gsem0, gsem1)
    c = lax.axis_index("c")
    s = lax.axis_index("s")
    wid = s * NC + c

    off = pl.multiple_of(s * ROWS_A, 8)
    pltpu.sync_copy(zeros_hbm.at[pl.ds(off, ROWS_A)],
                    acc_sh.at[pl.ds(off, ROWS_A)])

    @pl.when(s == NS - 1)
    def _zero_tail():
        pltpu.sync_copy(zeros_hbm.at[pl.ds(NS * ROWS_A, ROWS_REM)],
                        acc_sh.at[pl.ds(NS * ROWS_A, ROWS_REM)])

    pltpu.sync_copy(src_hbm.at[wid, 0], src_v)
    pltpu.sync_copy(dst_hbm.at[wid, 0], dst_v)
    plsc.subcore_barrier()

    def gather(q, b):
        return pltpu.async_copy(feat_hbm.at[src_v.at[q]], rows_v.at[b],
                                gsems[b])

    def wait_gather(b):
        pltpu.make_async_copy(feat_hbm.at[src_v.at[0]], rows_v.at[b],
                              gsems[b]).wait()

    def scatter(q, b):
        pltpu.sync_copy(rows_v.at[b], acc_sh.at[dst_v.at[q]], add=True)

    for half in range(2):
        if half == 1:
            pltpu.sync_copy(src_hbm.at[wid, 1], src_v)
            pltpu.sync_copy(dst_hbm.at[wid, 1], dst_v)
        gather(0, 0)

        def body(t, carry):
            gather(2 * t + 1, 1)
            wait_gather(0)
            scatter(2 * t, 0)
            gather(2 * t + 2, 0)
            wait_gather(1)
            scatter(2 * t + 1, 1)
            return carry

        lax.fori_loop(0, HALF // 2 - 1, body, 0)
        gather(HALF - 1, 1)
        wait_gather(0)
        scatter(HALF - 2, 0)
        wait_gather(1)
        scatter(HALF - 1, 1)

    plsc.subcore_barrier()
    pltpu.sync_copy(acc_sh.at[pl.ds(off, ROWS_A)],
                    out_hbm.at[c, pl.ds(off, ROWS_A)])

    @pl.when(s == NS - 1)
    def _out_tail():
        pltpu.sync_copy(acc_sh.at[pl.ds(NS * ROWS_A, ROWS_REM)],
                        out_hbm.at[c, pl.ds(NS * ROWS_A, ROWS_REM)])


_agg = pl.kernel(
    _agg_body,
    out_type=jax.ShapeDtypeStruct((NC, N, F), jnp.float32),
    mesh=plsc.VectorSubcoreMesh(core_axis_name="c", subcore_axis_name="s"),
    scratch_types=[
        pltpu.VMEM((HALF, CHUNK), jnp.int32),
        pltpu.VMEM((HALF, CHUNK), jnp.int32),
        pltpu.VMEM((2, CHUNK, F), jnp.float32),
        pltpu.VMEM_SHARED((N, F), jnp.float32),
        pltpu.SemaphoreType.DMA,
        pltpu.SemaphoreType.DMA,
    ],
)

FW = 32

_agg_w = pl.kernel(
    _agg_body,
    out_type=jax.ShapeDtypeStruct((NC, N, FW), jnp.float32),
    mesh=plsc.VectorSubcoreMesh(core_axis_name="c", subcore_axis_name="s"),
    compiler_params=pltpu.CompilerParams(use_tc_tiling_on_sc=False),
    scratch_types=[
        pltpu.VMEM((HALF, CHUNK), jnp.int32),
        pltpu.VMEM((HALF, CHUNK), jnp.int32),
        pltpu.VMEM((2, CHUNK, FW), jnp.float32),
        pltpu.VMEM_SHARED((N, FW), jnp.float32),
        pltpu.SemaphoreType.DMA,
        pltpu.SemaphoreType.DMA,
    ],
)


_ROWS = 1000
_GRID = N // _ROWS


def _lin_relu_tc(p_ref, w_ref, b_ref, o_ref):
    a = p_ref[0] + p_ref[1]
    z = lax.dot_general(a, w_ref[...], (((1,), (1,)), ((), ())),
                        preferred_element_type=jnp.float32)
    o_ref[...] = jnp.maximum(z + b_ref[...], 0.0)


def _lstm_tc(p_ref, h_ref, c_ref, wih_ref, whh_ref, b_ref, ho_ref, co_ref):
    a = p_ref[0] + p_ref[1]
    g = (lax.dot_general(a, wih_ref[...], (((1,), (1,)), ((), ())),
                         preferred_element_type=jnp.float32)
         + lax.dot_general(h_ref[...], whh_ref[...], (((1,), (1,)), ((), ())),
                           preferred_element_type=jnp.float32)
         + b_ref[...])
    i = jax.nn.sigmoid(g[:, 0:H])
    f = jax.nn.sigmoid(g[:, H:2 * H])
    gg = jnp.tanh(g[:, 2 * H:3 * H])
    o = jax.nn.sigmoid(g[:, 3 * H:4 * H])
    cc = f * c_ref[...] + i * gg
    ho_ref[...] = o * jnp.tanh(cc)
    co_ref[...] = cc


def _yvec_tc(h_ref, w_ref, o_ref):
    o_ref[...] = lax.dot_general(h_ref[...], w_ref[...],
                                 (((1,), (1,)), ((), ())),
                                 preferred_element_type=jnp.float32)


def _fin_tc(p_ref, b_ref, o_ref):
    o_ref[...] = p_ref[0] + p_ref[1] + b_ref[0]


_lin_relu = pl.pallas_call(
    _lin_relu_tc,
    grid=(_GRID,),
    in_specs=[
        pl.BlockSpec((2, _ROWS, F), lambda i: (0, i, 0)),
        pl.BlockSpec((H, F), lambda i: (0, 0)),
        pl.BlockSpec((1, H), lambda i: (0, 0)),
    ],
    out_specs=pl.BlockSpec((_ROWS, H), lambda i: (i, 0)),
    out_shape=jax.ShapeDtypeStruct((N, H), jnp.float32),
)

_lstm = pl.pallas_call(
    _lstm_tc,
    grid=(_GRID,),
    in_specs=[
        pl.BlockSpec((2, _ROWS, H), lambda i: (0, i, 0)),
        pl.BlockSpec((_ROWS, H), lambda i: (i, 0)),
        pl.BlockSpec((_ROWS, H), lambda i: (i, 0)),
        pl.BlockSpec((4 * H, H), lambda i: (0, 0)),
        pl.BlockSpec((4 * H, H), lambda i: (0, 0)),
        pl.BlockSpec((1, 4 * H), lambda i: (0, 0)),
    ],
    out_specs=[
        pl.BlockSpec((_ROWS, H), lambda i: (i, 0)),
        pl.BlockSpec((_ROWS, H), lambda i: (i, 0)),
    ],
    out_shape=[
        jax.ShapeDtypeStruct((N, H), jnp.float32),
        jax.ShapeDtypeStruct((N, H), jnp.float32),
    ],
)

_yvec = pl.pallas_call(
    _yvec_tc,
    grid=(_GRID,),
    in_specs=[
        pl.BlockSpec((_ROWS, H), lambda i: (i, 0)),
        pl.BlockSpec((128, H), lambda i: (0, 0)),
    ],
    out_specs=pl.BlockSpec((_ROWS, 128), lambda i: (i, 0)),
    out_shape=jax.ShapeDtypeStruct((N, 128), jnp.float32),
)

_fin = pl.pallas_call(
    _fin_tc,
    grid=(_GRID,),
    in_specs=[
        pl.BlockSpec((2, _ROWS, FW), lambda i: (0, i, 0)),
        pl.BlockSpec(memory_space=pltpu.SMEM),
    ],
    out_specs=pl.BlockSpec((_ROWS, FW), lambda i: (i, 0)),
    out_shape=jax.ShapeDtypeStruct((N, FW), jnp.float32),
)


def kernel(features, edge_index, W_in, b_in, W_ih, W_hh, b_ih, b_hh, W_out, b_out):
    src = edge_index[0].reshape(NW, 2, HALF, CHUNK)
    dst = edge_index[1].reshape(NW, 2, HALF, CHUNK)
    zeros = jnp.zeros((N, F), jnp.float32)
    b_in2 = b_in.reshape(1, H)
    b_g = (b_ih + b_hh).reshape(1, 4 * H)
    W_out_pad = jnp.zeros((128, H), jnp.float32).at[0].set(W_out[0])

    p = _agg(features, src, dst, zeros)
    h = _lin_relu(p, W_in, b_in2)

    h_t = jnp.zeros((N, H), jnp.float32)
    c_t = jnp.zeros((N, H), jnp.float32)

    p = _agg(h, src, dst, zeros)
    h_t, c_t = _lstm(p, h_t, c_t, W_ih, W_hh, b_g)
    for _ in range(DEPTH_ITERS):
        p = _agg(h_t, src, dst, zeros)
        h_t, c_t = _lstm(p, h_t, c_t, W_ih, W_hh, b_g)

    y32 = _yvec(h_t, W_out_pad)[:, :FW]
    p32 = _agg_w(y32, src, dst, jnp.zeros((N, FW), jnp.float32))
    return _fin(p32, b_out)[:, :1]
